# baseline (speedup 1.0000x reference)
import functools

import jax
import jax.numpy as jnp
from jax import lax
from jax.experimental import pallas as pl
from jax.experimental.pallas import tpu as pltpu
from jax.experimental.pallas import tpu_sc as plsc

NUM_ACTIONS = 100000
EMBED_DIM = 32
HIDDEN_DIM = 64
LATENT_DIM = 64
BATCH = 16384

NC = 2
NS = 16
NW = NC * NS
B_PER_W = BATCH // NW
CHUNK = 128
NCHUNK = B_PER_W // CHUNK

_sc_mesh = plsc.VectorSubcoreMesh(core_axis_name="c", subcore_axis_name="s")


@functools.partial(
    pl.kernel,
    mesh=_sc_mesh,
    out_type=jax.ShapeDtypeStruct((BATCH, EMBED_DIM), jnp.float32),
    scratch_types=[
        pltpu.VMEM((NCHUNK, CHUNK), jnp.int32),
        pltpu.VMEM((B_PER_W, EMBED_DIM), jnp.float32),
        pltpu.SemaphoreType.DMA,
    ],
    compiler_params=pltpu.CompilerParams(use_tc_tiling_on_sc=False),
)
def _sc_gather(idx_hbm, table_hbm, out_hbm, idx_v, rows_v, sem):
    wid = lax.axis_index("s") * NC + lax.axis_index("c")
    base = wid * B_PER_W
    for j in range(NCHUNK):
        pltpu.sync_copy(idx_hbm.at[pl.ds(base + j * CHUNK, CHUNK)],
                        idx_v.at[j])
    copies = []
    for j in range(NCHUNK):
        copies.append(
            pltpu.async_copy(
                table_hbm.at[idx_v.at[j]],
                rows_v.at[pl.ds(j * CHUNK, CHUNK)],
                sem,
            )
        )
    for c in copies:
        c.wait()
    pltpu.sync_copy(rows_v, out_hbm.at[pl.ds(base, B_PER_W)])


BLK = 2048
GRID = BATCH // BLK


def _mlp_body(e_ref, w1_ref, b1_ref, gamma_ref, beta_ref, w2_ref, b2_ref,
              out_ref):
    e = e_ref[...]
    h = jnp.dot(e, w1_ref[...], preferred_element_type=jnp.float32)
    h = h + b1_ref[...]
    mean = jnp.mean(h, axis=-1, keepdims=True)
    cen = h - mean
    var = jnp.mean(cen * cen, axis=-1, keepdims=True)
    h = cen * lax.rsqrt(var + 1e-5) * gamma_ref[...] + beta_ref[...]
    h = jnp.maximum(h, 0.0)
    z = jnp.dot(h, w2_ref[...], preferred_element_type=jnp.float32)
    out_ref[...] = z + b2_ref[...]


_mlp = pl.pallas_call(
    _mlp_body,
    grid=(GRID,),
    in_specs=[
        pl.BlockSpec((BLK, EMBED_DIM), lambda i: (i, 0)),
        pl.BlockSpec((EMBED_DIM, HIDDEN_DIM), lambda i: (0, 0)),
        pl.BlockSpec((1, HIDDEN_DIM), lambda i: (0, 0)),
        pl.BlockSpec((1, HIDDEN_DIM), lambda i: (0, 0)),
        pl.BlockSpec((1, HIDDEN_DIM), lambda i: (0, 0)),
        pl.BlockSpec((HIDDEN_DIM, LATENT_DIM), lambda i: (0, 0)),
        pl.BlockSpec((1, LATENT_DIM), lambda i: (0, 0)),
    ],
    out_specs=pl.BlockSpec((BLK, LATENT_DIM), lambda i: (i, 0)),
    out_shape=jax.ShapeDtypeStruct((BATCH, LATENT_DIM), jnp.float32),
)


def kernel(x, table, W1, b1, gamma, beta, W2, b2):
    e = _sc_gather(x.astype(jnp.int32), table)
    return _mlp(
        e,
        W1,
        b1.reshape(1, HIDDEN_DIM),
        gamma.reshape(1, HIDDEN_DIM),
        beta.reshape(1, HIDDEN_DIM),
        W2,
        b2.reshape(1, LATENT_DIM),
    )


# speedup vs baseline: 1.9268x; 1.9268x over previous
import functools

import jax
import jax.numpy as jnp
from jax import lax
from jax.experimental import pallas as pl
from jax.experimental.pallas import tpu as pltpu
from jax.experimental.pallas import tpu_sc as plsc

NUM_ACTIONS = 100000
EMBED_DIM = 32
HIDDEN_DIM = 64
LATENT_DIM = 64
BATCH = 16384

NC = 2
NS = 16
CH = 2048

_sc_mesh = plsc.VectorSubcoreMesh(core_axis_name="c", subcore_axis_name="s")


@functools.partial(
    pl.kernel,
    mesh=_sc_mesh,
    out_type=jax.ShapeDtypeStruct((EMBED_DIM, BATCH), jnp.float32),
    scratch_types=[
        pltpu.VMEM((NUM_ACTIONS,), jnp.float32),
        pltpu.VMEM((CH,), jnp.int32),
        pltpu.VMEM((CH,), jnp.float32),
    ],
    compiler_params=pltpu.CompilerParams(
        use_tc_tiling_on_sc=True, needs_layout_passes=False),
)
def _sc_gather(table_t, idx_hbm, out_t, slab_v, idx_v, g_v):
    wid = lax.axis_index("s") * NC + lax.axis_index("c")
    pltpu.sync_copy(table_t.at[wid], slab_v)

    def chunk(ci, _):
        pltpu.sync_copy(idx_hbm.at[pl.ds(ci * CH, CH)], idx_v)

        def sub(i, _):
            iv = idx_v[pl.ds(i * 16, 16)]
            g_v[pl.ds(i * 16, 16)] = plsc.load_gather(slab_v, [iv])
            return ()

        lax.fori_loop(0, CH // 16, sub, (), unroll=8)
        pltpu.sync_copy(g_v, out_t.at[wid, pl.ds(ci * CH, CH)])
        return ()

    lax.fori_loop(0, BATCH // CH, chunk, (), unroll=1)


BLK = 2048
GRID = BATCH // BLK


def _mlp_body(et_ref, w1_ref, w2_ref, b1_ref, gamma_ref, beta_ref, b2_ref,
              out_ref):
    et = et_ref[...]
    ht = lax.dot_general(w1_ref[...], et, (((0,), (0,)), ((), ())),
                         preferred_element_type=jnp.float32)
    ht = ht + b1_ref[...]
    mean = jnp.mean(ht, axis=0, keepdims=True)
    cen = ht - mean
    var = jnp.mean(cen * cen, axis=0, keepdims=True)
    ht = cen * lax.rsqrt(var + 1e-5) * gamma_ref[...] + beta_ref[...]
    ht = jnp.maximum(ht, 0.0)
    zt = lax.dot_general(w2_ref[...], ht, (((0,), (0,)), ((), ())),
                         preferred_element_type=jnp.float32)
    out_ref[...] = zt + b2_ref[...]


_mlp = pl.pallas_call(
    _mlp_body,
    grid=(GRID,),
    in_specs=[
        pl.BlockSpec((EMBED_DIM, BLK), lambda i: (0, i)),
        pl.BlockSpec((EMBED_DIM, HIDDEN_DIM), lambda i: (0, 0)),
        pl.BlockSpec((HIDDEN_DIM, LATENT_DIM), lambda i: (0, 0)),
        pl.BlockSpec((HIDDEN_DIM, 1), lambda i: (0, 0)),
        pl.BlockSpec((HIDDEN_DIM, 1), lambda i: (0, 0)),
        pl.BlockSpec((HIDDEN_DIM, 1), lambda i: (0, 0)),
        pl.BlockSpec((LATENT_DIM, 1), lambda i: (0, 0)),
    ],
    out_specs=pl.BlockSpec((LATENT_DIM, BLK), lambda i: (0, i)),
    out_shape=jax.ShapeDtypeStruct((LATENT_DIM, BATCH), jnp.float32),
)


def kernel(x, table, W1, b1, gamma, beta, W2, b2):
    et = _sc_gather(table.T, x.astype(jnp.int32))
    zt = _mlp(
        et,
        W1,
        W2,
        b1.reshape(HIDDEN_DIM, 1),
        gamma.reshape(HIDDEN_DIM, 1),
        beta.reshape(HIDDEN_DIM, 1),
        b2.reshape(LATENT_DIM, 1),
    )
    return zt.T


# speedup vs baseline: 2.1582x; 1.1201x over previous
import functools

import jax
import jax.numpy as jnp
from jax import lax
from jax.experimental import pallas as pl
from jax.experimental.pallas import tpu as pltpu
from jax.experimental.pallas import tpu_sc as plsc

NUM_ACTIONS = 100000
EMBED_DIM = 32
HIDDEN_DIM = 64
LATENT_DIM = 64
BATCH = 16384

NC = 2
NS = 16
CH = 4096
NCH = BATCH // CH

_sc_mesh = plsc.VectorSubcoreMesh(core_axis_name="c", subcore_axis_name="s")


@functools.partial(
    pl.kernel,
    mesh=_sc_mesh,
    out_type=jax.ShapeDtypeStruct((EMBED_DIM, BATCH), jnp.float32),
    scratch_types=[
        pltpu.VMEM((NUM_ACTIONS,), jnp.float32),
        pltpu.VMEM((BATCH,), jnp.int32),
        pltpu.VMEM((2, CH), jnp.float32),
        pltpu.SemaphoreType.DMA,
        pltpu.SemaphoreType.DMA,
    ],
    compiler_params=pltpu.CompilerParams(
        use_tc_tiling_on_sc=True, needs_layout_passes=False),
)
def _sc_gather(table_t, idx_hbm, out_t, slab_v, idx_v, g_v, sem_slab,
               sem_out):
    wid = lax.axis_index("s") * NC + lax.axis_index("c")
    slab_cp = pltpu.async_copy(table_t.at[wid], slab_v, sem_slab)
    pltpu.sync_copy(idx_hbm, idx_v)
    slab_cp.wait()

    out_cps = []
    for ci in range(NCH):
        buf = ci % 2

        def sub(i, _, ci=ci, buf=buf):
            iv = idx_v[pl.ds(ci * CH + i * 16, 16)]
            g_v[buf, pl.ds(i * 16, 16)] = plsc.load_gather(slab_v, [iv])
            return ()

        if ci >= 2:
            out_cps[ci - 2].wait()
        lax.fori_loop(0, CH // 16, sub, (), unroll=16)
        out_cps.append(
            pltpu.async_copy(
                g_v.at[buf], out_t.at[wid, pl.ds(ci * CH, CH)], sem_out))
    for cp in out_cps[-2:]:
        cp.wait()


BLK = 2048
GRID = BATCH // BLK


def _mlp_body(et_ref, w1_ref, w2_ref, b1_ref, gamma_ref, beta_ref, b2_ref,
              out_ref):
    et = et_ref[...]
    ht = lax.dot_general(w1_ref[...], et, (((0,), (0,)), ((), ())),
                         preferred_element_type=jnp.float32)
    ht = ht + b1_ref[...]
    mean = jnp.mean(ht, axis=0, keepdims=True)
    cen = ht - mean
    var = jnp.mean(cen * cen, axis=0, keepdims=True)
    ht = cen * lax.rsqrt(var + 1e-5) * gamma_ref[...] + beta_ref[...]
    ht = jnp.maximum(ht, 0.0)
    zt = lax.dot_general(w2_ref[...], ht, (((0,), (0,)), ((), ())),
                         preferred_element_type=jnp.float32)
    out_ref[...] = zt + b2_ref[...]


_mlp = pl.pallas_call(
    _mlp_body,
    grid=(GRID,),
    in_specs=[
        pl.BlockSpec((EMBED_DIM, BLK), lambda i: (0, i)),
        pl.BlockSpec((EMBED_DIM, HIDDEN_DIM), lambda i: (0, 0)),
        pl.BlockSpec((HIDDEN_DIM, LATENT_DIM), lambda i: (0, 0)),
        pl.BlockSpec((HIDDEN_DIM, 1), lambda i: (0, 0)),
        pl.BlockSpec((HIDDEN_DIM, 1), lambda i: (0, 0)),
        pl.BlockSpec((HIDDEN_DIM, 1), lambda i: (0, 0)),
        pl.BlockSpec((LATENT_DIM, 1), lambda i: (0, 0)),
    ],
    out_specs=pl.BlockSpec((LATENT_DIM, BLK), lambda i: (0, i)),
    out_shape=jax.ShapeDtypeStruct((LATENT_DIM, BATCH), jnp.float32),
)


def kernel(x, table, W1, b1, gamma, beta, W2, b2):
    et = _sc_gather(table.T, x.astype(jnp.int32))
    zt = _mlp(
        et,
        W1,
        W2,
        b1.reshape(HIDDEN_DIM, 1),
        gamma.reshape(HIDDEN_DIM, 1),
        beta.reshape(HIDDEN_DIM, 1),
        b2.reshape(LATENT_DIM, 1),
    )
    return zt.T


# speedup vs baseline: 2.5826x; 1.1966x over previous
import functools

import jax
import jax.numpy as jnp
from jax import lax
from jax.experimental import pallas as pl
from jax.experimental.pallas import tpu as pltpu
from jax.experimental.pallas import tpu_sc as plsc

NUM_ACTIONS = 100000
EMBED_DIM = 32
HIDDEN_DIM = 64
LATENT_DIM = 64
BATCH = 16384

NC = 2
NS = 16
CH = 4096
NCH = BATCH // CH

_sc_mesh = plsc.VectorSubcoreMesh(core_axis_name="c", subcore_axis_name="s")


@functools.partial(
    pl.kernel,
    mesh=_sc_mesh,
    out_type=jax.ShapeDtypeStruct((EMBED_DIM, BATCH), jnp.float32),
    scratch_types=[
        pltpu.VMEM((NUM_ACTIONS,), jnp.float32),
        pltpu.VMEM((BATCH,), jnp.int32),
        pltpu.VMEM((2, CH), jnp.float32),
        pltpu.SemaphoreType.DMA,
        pltpu.SemaphoreType.DMA,
    ],
    compiler_params=pltpu.CompilerParams(
        use_tc_tiling_on_sc=True, needs_layout_passes=False),
)
def _sc_gather(table_t, idx_hbm, out_t, slab_v, idx_v, g_v, sem_slab,
               sem_out):
    wid = lax.axis_index("s") * NC + lax.axis_index("c")
    slab_cp = pltpu.async_copy(table_t.at[wid], slab_v, sem_slab)
    pltpu.sync_copy(idx_hbm, idx_v)
    slab_cp.wait()

    out_cps = []
    for ci in range(NCH):
        buf = ci % 2

        if ci >= 2:
            out_cps[ci - 2].wait()

        @plsc.parallel_loop(0, CH, step=16, unroll=8)
        def _(i, ci=ci, buf=buf):
            iv = idx_v[pl.ds(ci * CH + i, 16)]
            g_v[buf, pl.ds(i, 16)] = plsc.load_gather(slab_v, [iv])
        out_cps.append(
            pltpu.async_copy(
                g_v.at[buf], out_t.at[wid, pl.ds(ci * CH, CH)], sem_out))
    for cp in out_cps[-2:]:
        cp.wait()


BLK = 2048
GRID = BATCH // BLK


def _mlp_body(et_ref, w1_ref, w2_ref, b1_ref, gamma_ref, beta_ref, b2_ref,
              out_ref):
    et = et_ref[...]
    ht = lax.dot_general(w1_ref[...], et, (((0,), (0,)), ((), ())),
                         preferred_element_type=jnp.float32)
    ht = ht + b1_ref[...]
    mean = jnp.mean(ht, axis=0, keepdims=True)
    cen = ht - mean
    var = jnp.mean(cen * cen, axis=0, keepdims=True)
    ht = cen * lax.rsqrt(var + 1e-5) * gamma_ref[...] + beta_ref[...]
    ht = jnp.maximum(ht, 0.0)
    zt = lax.dot_general(w2_ref[...], ht, (((0,), (0,)), ((), ())),
                         preferred_element_type=jnp.float32)
    out_ref[...] = zt + b2_ref[...]


_mlp = pl.pallas_call(
    _mlp_body,
    grid=(GRID,),
    in_specs=[
        pl.BlockSpec((EMBED_DIM, BLK), lambda i: (0, i)),
        pl.BlockSpec((EMBED_DIM, HIDDEN_DIM), lambda i: (0, 0)),
        pl.BlockSpec((HIDDEN_DIM, LATENT_DIM), lambda i: (0, 0)),
        pl.BlockSpec((HIDDEN_DIM, 1), lambda i: (0, 0)),
        pl.BlockSpec((HIDDEN_DIM, 1), lambda i: (0, 0)),
        pl.BlockSpec((HIDDEN_DIM, 1), lambda i: (0, 0)),
        pl.BlockSpec((LATENT_DIM, 1), lambda i: (0, 0)),
    ],
    out_specs=pl.BlockSpec((LATENT_DIM, BLK), lambda i: (0, i)),
    out_shape=jax.ShapeDtypeStruct((LATENT_DIM, BATCH), jnp.float32),
)


def kernel(x, table, W1, b1, gamma, beta, W2, b2):
    et = _sc_gather(table.T, x.astype(jnp.int32))
    zt = _mlp(
        et,
        W1,
        W2,
        b1.reshape(HIDDEN_DIM, 1),
        gamma.reshape(HIDDEN_DIM, 1),
        beta.reshape(HIDDEN_DIM, 1),
        b2.reshape(LATENT_DIM, 1),
    )
    return zt.T


# speedup vs baseline: 2.7540x; 1.0663x over previous
import functools

import jax
import jax.numpy as jnp
from jax import lax
from jax.experimental import pallas as pl
from jax.experimental.pallas import tpu as pltpu
from jax.experimental.pallas import tpu_sc as plsc

NUM_ACTIONS = 100000
EMBED_DIM = 32
HIDDEN_DIM = 64
LATENT_DIM = 64
BATCH = 16384

NC = 2
NS = 16
CH = 4096
NCH = BATCH // CH

_sc_mesh = plsc.VectorSubcoreMesh(core_axis_name="c", subcore_axis_name="s")


@functools.partial(
    pl.kernel,
    mesh=_sc_mesh,
    out_type=jax.ShapeDtypeStruct((EMBED_DIM, BATCH), jnp.float32),
    scratch_types=[
        pltpu.VMEM((NUM_ACTIONS,), jnp.float32),
        pltpu.VMEM((BATCH,), jnp.int32),
        pltpu.VMEM((2, CH), jnp.float32),
        pltpu.SemaphoreType.DMA,
        pltpu.SemaphoreType.DMA,
    ],
    compiler_params=pltpu.CompilerParams(
        use_tc_tiling_on_sc=True, needs_layout_passes=False),
)
def _sc_gather(table_t, idx_hbm, out_t, slab_v, idx_v, g_v, sem_slab,
               sem_out):
    wid = lax.axis_index("s") * NC + lax.axis_index("c")
    slab_cp = pltpu.async_copy(table_t.at[wid], slab_v, sem_slab)
    pltpu.sync_copy(idx_hbm, idx_v)
    slab_cp.wait()

    out_cps = []
    for ci in range(NCH):
        buf = ci % 2

        if ci >= 2:
            out_cps[ci - 2].wait()

        @plsc.parallel_loop(0, CH, step=16, unroll=8)
        def _(i, ci=ci, buf=buf):
            iv = idx_v[pl.ds(ci * CH + i, 16)]
            g_v[buf, pl.ds(i, 16)] = plsc.load_gather(slab_v, [iv])
        out_cps.append(
            pltpu.async_copy(
                g_v.at[buf], out_t.at[wid, pl.ds(ci * CH, CH)], sem_out))
    for cp in out_cps[-2:]:
        cp.wait()


BLK = 4096
GRID = BATCH // BLK


def _mlp_body(et_ref, w1_ref, w2_ref, b1_ref, gamma_ref, beta_ref, b2_ref,
              out_ref):
    et = et_ref[...]
    ht = lax.dot_general(w1_ref[...], et, (((0,), (0,)), ((), ())),
                         preferred_element_type=jnp.float32)
    ht = ht + b1_ref[...]
    mean = jnp.mean(ht, axis=0, keepdims=True)
    cen = ht - mean
    var = jnp.mean(cen * cen, axis=0, keepdims=True)
    ht = cen * lax.rsqrt(var + 1e-5) * gamma_ref[...] + beta_ref[...]
    ht = jnp.maximum(ht, 0.0)
    zt = lax.dot_general(w2_ref[...], ht, (((0,), (0,)), ((), ())),
                         preferred_element_type=jnp.float32)
    out_ref[...] = zt + b2_ref[...]


_mlp = pl.pallas_call(
    _mlp_body,
    grid=(GRID,),
    in_specs=[
        pl.BlockSpec((EMBED_DIM, BLK), lambda i: (0, i)),
        pl.BlockSpec((EMBED_DIM, HIDDEN_DIM), lambda i: (0, 0)),
        pl.BlockSpec((HIDDEN_DIM, LATENT_DIM), lambda i: (0, 0)),
        pl.BlockSpec((HIDDEN_DIM, 1), lambda i: (0, 0)),
        pl.BlockSpec((HIDDEN_DIM, 1), lambda i: (0, 0)),
        pl.BlockSpec((HIDDEN_DIM, 1), lambda i: (0, 0)),
        pl.BlockSpec((LATENT_DIM, 1), lambda i: (0, 0)),
    ],
    out_specs=pl.BlockSpec((LATENT_DIM, BLK), lambda i: (0, i)),
    out_shape=jax.ShapeDtypeStruct((LATENT_DIM, BATCH), jnp.float32),
)


def kernel(x, table, W1, b1, gamma, beta, W2, b2):
    et = _sc_gather(table.T, x.astype(jnp.int32))
    zt = _mlp(
        et,
        W1,
        W2,
        b1.reshape(HIDDEN_DIM, 1),
        gamma.reshape(HIDDEN_DIM, 1),
        beta.reshape(HIDDEN_DIM, 1),
        b2.reshape(LATENT_DIM, 1),
    )
    return zt.T


# speedup vs baseline: 2.7702x; 1.0059x over previous
import functools

import jax
import jax.numpy as jnp
from jax import lax
from jax.experimental import pallas as pl
from jax.experimental.pallas import tpu as pltpu
from jax.experimental.pallas import tpu_sc as plsc

NUM_ACTIONS = 100000
EMBED_DIM = 32
HIDDEN_DIM = 64
LATENT_DIM = 64
BATCH = 16384

NC = 2
NS = 16
CH = 4096
NCH = BATCH // CH

_sc_mesh = plsc.VectorSubcoreMesh(core_axis_name="c", subcore_axis_name="s")


@functools.partial(
    pl.kernel,
    mesh=_sc_mesh,
    out_type=jax.ShapeDtypeStruct((EMBED_DIM, BATCH), jnp.float32),
    scratch_types=[
        pltpu.VMEM((NUM_ACTIONS,), jnp.float32),
        pltpu.VMEM((BATCH,), jnp.int32),
        pltpu.VMEM((2, CH), jnp.float32),
        pltpu.SemaphoreType.DMA,
        pltpu.SemaphoreType.DMA,
    ],
    compiler_params=pltpu.CompilerParams(
        use_tc_tiling_on_sc=True, needs_layout_passes=False),
)
def _sc_gather(table_t, idx_hbm, out_t, slab_v, idx_v, g_v, sem_slab,
               sem_out):
    wid = lax.axis_index("s") * NC + lax.axis_index("c")
    slab_cp = pltpu.async_copy(table_t.at[wid], slab_v, sem_slab)
    pltpu.sync_copy(idx_hbm, idx_v)
    slab_cp.wait()

    out_cps = []
    for ci in range(NCH):
        buf = ci % 2

        if ci >= 2:
            out_cps[ci - 2].wait()

        @plsc.parallel_loop(0, CH, step=16, unroll=16)
        def _(i, ci=ci, buf=buf):
            iv = idx_v[pl.ds(ci * CH + i, 16)]
            g_v[buf, pl.ds(i, 16)] = plsc.load_gather(slab_v, [iv])
        out_cps.append(
            pltpu.async_copy(
                g_v.at[buf], out_t.at[wid, pl.ds(ci * CH, CH)], sem_out))
    for cp in out_cps[-2:]:
        cp.wait()


BLK = 8192
GRID = BATCH // BLK


def _mlp_body(et_ref, w1_ref, w2_ref, b1_ref, gamma_ref, beta_ref, b2_ref,
              out_ref):
    et = et_ref[...]
    ht = lax.dot_general(w1_ref[...], et, (((0,), (0,)), ((), ())),
                         preferred_element_type=jnp.float32)
    ht = ht + b1_ref[...]
    mean = jnp.mean(ht, axis=0, keepdims=True)
    cen = ht - mean
    var = jnp.mean(cen * cen, axis=0, keepdims=True)
    ht = cen * lax.rsqrt(var + 1e-5) * gamma_ref[...] + beta_ref[...]
    ht = jnp.maximum(ht, 0.0)
    zt = lax.dot_general(w2_ref[...], ht, (((0,), (0,)), ((), ())),
                         preferred_element_type=jnp.float32)
    out_ref[...] = zt + b2_ref[...]


_mlp = pl.pallas_call(
    _mlp_body,
    grid=(GRID,),
    in_specs=[
        pl.BlockSpec((EMBED_DIM, BLK), lambda i: (0, i)),
        pl.BlockSpec((EMBED_DIM, HIDDEN_DIM), lambda i: (0, 0)),
        pl.BlockSpec((HIDDEN_DIM, LATENT_DIM), lambda i: (0, 0)),
        pl.BlockSpec((HIDDEN_DIM, 1), lambda i: (0, 0)),
        pl.BlockSpec((HIDDEN_DIM, 1), lambda i: (0, 0)),
        pl.BlockSpec((HIDDEN_DIM, 1), lambda i: (0, 0)),
        pl.BlockSpec((LATENT_DIM, 1), lambda i: (0, 0)),
    ],
    out_specs=pl.BlockSpec((LATENT_DIM, BLK), lambda i: (0, i)),
    out_shape=jax.ShapeDtypeStruct((LATENT_DIM, BATCH), jnp.float32),
)


def kernel(x, table, W1, b1, gamma, beta, W2, b2):
    et = _sc_gather(table.T, x.astype(jnp.int32))
    zt = _mlp(
        et,
        W1,
        W2,
        b1.reshape(HIDDEN_DIM, 1),
        gamma.reshape(HIDDEN_DIM, 1),
        beta.reshape(HIDDEN_DIM, 1),
        b2.reshape(LATENT_DIM, 1),
    )
    return zt.T
